# baseline (speedup 1.0000x reference)
import jax
import jax.numpy as jnp
from jax.experimental import pallas as pl
from jax.experimental.pallas import tpu as pltpu

N = 2048
NINP = 512
NHID = 256
HEADS = 8
NOUT = 256
R = 512
NB = N // R
CC = 512
LOG2E = 1.4426950408889634


def _pre1_kernel(x_ref, w_ref, a1s_ref, a1d_ref, wh_ref, sd_ref):
    x = x_ref[...]
    log2e = jnp.float32(LOG2E)
    for h in range(HEADS):
        whh = jnp.dot(x, w_ref[:, h * NHID:(h + 1) * NHID],
                      preferred_element_type=jnp.float32)
        wh_ref[h, :, :] = whh.astype(jnp.bfloat16)
        a1c = jnp.stack([a1s_ref[h, :], a1d_ref[h, :]], axis=1)
        sdh = jnp.dot(whh, a1c, preferred_element_type=jnp.float32) * log2e
        sd_ref[h, :] = sdh[:, 0]
        sd_ref[HEADS + h, :] = sdh[:, 1]


def _attn12_kernel(adj_ref, wh_ref, sd_ref, w2_ref, a2s_ref, a2d_ref,
                   att1_ref, att2_ref, h2_ref,
                   wh2_ref, sd2_ref, whb_ref):
    h = pl.program_id(0)
    i = pl.program_id(1)

    @pl.when(h < HEADS)
    def _layer1():
        s = sd_ref[h, pl.ds(i * R, R)]
        sc = s[:, None]
        o = jnp.zeros((R, NHID), jnp.float32)
        tot = jnp.zeros((R, 1), jnp.float32)
        for c in range(N // CC):
            d = sd_ref[HEADS + h, pl.ds(c * CC, CC)]
            e = sc + d[None, :]
            e = jnp.maximum(e, 0.2 * e)
            pc = jnp.exp2(e) * adj_ref[pl.ds(i * R, R), pl.ds(c * CC, CC)]
            att1_ref[0, :, pl.ds(c * CC, CC)] = pc
            tot += jnp.sum(pc, axis=1, keepdims=True)
            o += jnp.dot(pc.astype(jnp.bfloat16),
                         wh_ref[0, pl.ds(c * CC, CC), :],
                         preferred_element_type=jnp.float32)
        r = 1.0 / tot
        att1_ref[0, :, :] *= r
        o = o * r
        o = jnp.where(o > 0, o, jnp.exp(jnp.minimum(o, 0.0)) - 1.0)
        w2h = w2_ref[pl.ds(h * NHID, NHID), :].astype(jnp.bfloat16)
        part = jnp.dot(o.astype(jnp.bfloat16), w2h,
                       preferred_element_type=jnp.float32)

        @pl.when(h == 0)
        def _():
            wh2_ref[pl.ds(i * R, R), :] = part

        @pl.when(h > 0)
        def _():
            wh2_ref[pl.ds(i * R, R), :] += part

    @pl.when(h == HEADS)
    def _layer2():
        @pl.when(i == 0)
        def _():
            a2c = jnp.stack([a2s_ref[0, :], a2d_ref[0, :]], axis=1)
            sd = jnp.dot(wh2_ref[...], a2c,
                         preferred_element_type=jnp.float32)
            sd2_ref[...] = sd.T * jnp.float32(LOG2E)
            whb_ref[...] = wh2_ref[...].astype(jnp.bfloat16)

        s = sd2_ref[0, pl.ds(i * R, R)]
        sc = s[:, None]
        o = jnp.zeros((R, NOUT), jnp.float32)
        tot = jnp.zeros((R, 1), jnp.float32)
        for c in range(N // CC):
            d = sd2_ref[1, pl.ds(c * CC, CC)]
            e = sc + d[None, :]
            e = jnp.maximum(e, 0.2 * e)
            pc = jnp.exp2(e) * adj_ref[pl.ds(i * R, R), pl.ds(c * CC, CC)]
            att2_ref[:, pl.ds(c * CC, CC)] = pc
            tot += jnp.sum(pc, axis=1, keepdims=True)
            o += jnp.dot(pc.astype(jnp.bfloat16),
                         whb_ref[pl.ds(c * CC, CC), :],
                         preferred_element_type=jnp.float32)
        r = 1.0 / tot
        att2_ref[...] *= r
        h2_ref[...] = o * r


def kernel(inp, adj, W1, a1_src, a1_dst, W2, a2_src, a2_dst):
    f32 = jnp.float32

    Wh1, sd1 = pl.pallas_call(
        _pre1_kernel,
        grid=(NB,),
        in_specs=[
            pl.BlockSpec((R, NINP), lambda i: (i, 0)),
            pl.BlockSpec((NINP, N), lambda i: (0, 0)),
            pl.BlockSpec((HEADS, NHID), lambda i: (0, 0)),
            pl.BlockSpec((HEADS, NHID), lambda i: (0, 0)),
        ],
        out_specs=[
            pl.BlockSpec((HEADS, R, NHID), lambda i: (0, i, 0)),
            pl.BlockSpec((2 * HEADS, R), lambda i: (0, i)),
        ],
        out_shape=[
            jax.ShapeDtypeStruct((HEADS, N, NHID), jnp.bfloat16),
            jax.ShapeDtypeStruct((2 * HEADS, N), f32),
        ],
    )(inp, W1, a1_src, a1_dst)

    h7 = HEADS - 1

    att1, att2, h2 = pl.pallas_call(
        _attn12_kernel,
        grid=(HEADS + 1, NB),
        in_specs=[
            pl.BlockSpec((N, N), lambda h, i: (0, 0)),
            pl.BlockSpec((1, N, NHID),
                         lambda h, i: (jnp.minimum(h, h7), 0, 0)),
            pl.BlockSpec((2 * HEADS, N), lambda h, i: (0, 0)),
            pl.BlockSpec((HEADS * NHID, NOUT), lambda h, i: (0, 0)),
            pl.BlockSpec((1, NOUT), lambda h, i: (0, 0)),
            pl.BlockSpec((1, NOUT), lambda h, i: (0, 0)),
        ],
        out_specs=[
            pl.BlockSpec((1, R, N),
                         lambda h, i: (jnp.minimum(h, h7),
                                       jnp.where(h < HEADS, i, NB - 1), 0)),
            pl.BlockSpec((R, N),
                         lambda h, i: (jnp.where(h < HEADS, 0, i), 0)),
            pl.BlockSpec((R, NOUT),
                         lambda h, i: (jnp.where(h < HEADS, 0, i), 0)),
        ],
        out_shape=[
            jax.ShapeDtypeStruct((HEADS, N, N), f32),
            jax.ShapeDtypeStruct((N, N), f32),
            jax.ShapeDtypeStruct((N, NOUT), f32),
        ],
        scratch_shapes=[
            pltpu.VMEM((N, NOUT), f32),
            pltpu.VMEM((2, N), f32),
            pltpu.VMEM((N, NOUT), jnp.bfloat16),
        ],
    )(adj, Wh1, sd1, W2, a2_src[None, :], a2_dst[None, :])

    return (h2, att1, att2)
